# baseline (speedup 1.0000x reference)
import functools

import jax
import jax.numpy as jnp
from jax import lax
from jax.experimental import pallas as pl
from jax.experimental.pallas import tpu as pltpu
from jax.experimental.pallas import tpu_sc as plsc

_B, _L = 4096, 50
_N = _B * _L
_ED = 32
_NT = 5
_NC, _NS = 2, 16
_NW = _NC * _NS
_BPW = _N // _NW
_CH = 80
_CPW = _BPW // _CH
_NP = 4
_NG = _CPW // _NP


def _build_sc_kernel():
    mesh = plsc.VectorSubcoreMesh(core_axis_name="c", subcore_axis_name="s")

    scratch = (
        [pltpu.VMEM((_CH,), jnp.int32) for _ in range(_NP * _NT)]
        + [pltpu.VMEM((_CH, _ED), jnp.float32) for _ in range(_NP * _NT)]
        + [pltpu.SemaphoreType.DMA for _ in range(3 * _NP)]
    )

    @functools.partial(
        pl.kernel,
        out_type=jax.ShapeDtypeStruct((_N, _NT * _ED), jnp.float32),
        mesh=mesh,
        scratch_types=scratch,
        compiler_params=pltpu.CompilerParams(use_tc_tiling_on_sc=False),
    )
    def k(i0, i1, i2, i3, i4, w0, w1, w2, w3, w4, out_hbm, *s):
        idx_hbm = (i0, i1, i2, i3, i4)
        tabs = (w0, w1, w2, w3, w4)
        iv = tuple(tuple(s[p * _NT + t] for t in range(_NT))
                   for p in range(_NP))
        rv = tuple(tuple(s[_NP * _NT + p * _NT + t] for t in range(_NT))
                   for p in range(_NP))
        si = s[2 * _NP * _NT: 2 * _NP * _NT + _NP]
        sg = s[2 * _NP * _NT + _NP: 2 * _NP * _NT + 2 * _NP]
        sw = s[2 * _NP * _NT + 2 * _NP: 2 * _NP * _NT + 3 * _NP]
        wid = lax.axis_index("s") * _NC + lax.axis_index("c")
        base0 = wid * _BPW

        def start_idx(p, c):
            for t in range(_NT):
                pltpu.make_async_copy(
                    idx_hbm[t].at[pl.ds(base0 + c * _CH, _CH)],
                    iv[p][t], si[p]).start()

        def wait_idx(p):
            for t in range(_NT):
                pltpu.make_async_copy(
                    idx_hbm[t].at[pl.ds(base0, _CH)], iv[p][t], si[p]).wait()

        def start_gather(p):
            for t in range(_NT):
                pltpu.make_async_copy(
                    tabs[t].at[iv[p][t]], rv[p][t], sg[p]).start()

        def wait_gather(p):
            for t in range(_NT):
                pltpu.make_async_copy(
                    tabs[t].at[iv[p][t]], rv[p][t], sg[p]).wait()

        def start_write(p, c):
            for t in range(_NT):
                pltpu.make_async_copy(
                    rv[p][t],
                    out_hbm.at[pl.ds(base0 + c * _CH, _CH),
                               pl.ds(t * _ED, _ED)],
                    sw[p]).start()

        def wait_write(p):
            for t in range(_NT):
                pltpu.make_async_copy(
                    rv[p][t],
                    out_hbm.at[pl.ds(base0, _CH), pl.ds(t * _ED, _ED)],
                    sw[p]).wait()

        for p in range(_NP):
            start_idx(p, p)
        wait_idx(0)
        start_gather(0)
        wait_idx(1)
        start_gather(1)

        def slot(q, c, first_group, last_group):
            wait_gather(q)
            start_write(q, c)
            q2 = (q + 2) % _NP
            if not last_group:
                if not (first_group and q < 2):
                    wait_write(q2)
                wait_idx(q2)
                start_gather(q2)
                start_idx(q, c + _NP)
            elif q < 2:
                wait_write(q2)
                wait_idx(q2)
                start_gather(q2)

        for q in range(_NP):
            slot(q, q, True, False)

        @pl.loop(1, _NG - 1)
        def _(g):
            c0 = g * _NP
            for q in range(_NP):
                slot(q, c0 + q, False, False)

        c0 = (_NG - 1) * _NP
        for q in range(_NP):
            slot(q, c0 + q, False, True)
        for p in range(_NP):
            wait_write(p)

    return k


_sc_gather = _build_sc_kernel()


def kernel(stage, p1_action, p1_character, p2_action, p2_character,
           W_stage, W_p1_action, W_p1_character, W_p2_action, W_p2_character):
    idxs = [x.reshape(_N) for x in
            (stage, p1_action, p1_character, p2_action, p2_character)]
    out = _sc_gather(*idxs, W_stage, W_p1_action, W_p1_character,
                     W_p2_action, W_p2_character)
    return out.reshape(_B, _L, _NT * _ED)
